# baseline (speedup 1.0000x reference)
import jax
import jax.numpy as jnp
import numpy as np
from jax.experimental import pallas as pl
from jax.experimental.pallas import tpu as pltpu

B = 4
N = 16384
C_FEAT = 16
NPOINT = 512
NR, NL = 128, 128
QB = 128
K = 32
THR1 = np.float32(0.2 * 0.2)
THR2 = np.float32(0.4 * 0.4)
MLPS = [[19, 32, 32], [19, 32, 64]]
NSAMPLES = [16, 32]


def _fps_kernel(far0_ref, pts_ref, ox_ref, oy_ref, oz_ref, dist_ref, acc_ref):
    X = pts_ref[0, 0]
    Y = pts_ref[0, 1]
    Z = pts_ref[0, 2]
    flat = (jax.lax.broadcasted_iota(jnp.int32, (NR, NL), 0) * NL
            + jax.lax.broadcasted_iota(jnp.int32, (NR, NL), 1))
    lane1 = jax.lax.broadcasted_iota(jnp.int32, (1, NL), 1)
    out_flat = (jax.lax.broadcasted_iota(jnp.int32, (4, 128), 0) * 128
                + jax.lax.broadcasted_iota(jnp.int32, (4, 128), 1))

    dist_ref[...] = jnp.full((NR, NL), jnp.inf, jnp.float32)
    acc_ref[...] = jnp.zeros((3, 4, 128), jnp.float32)

    def body(i, far):
        r = far // NL
        l = far % NL
        sel = lane1 == l
        cx = jnp.sum(jnp.where(sel, pts_ref[0, 0, pl.ds(r, 1), :], 0.0))
        cy = jnp.sum(jnp.where(sel, pts_ref[0, 1, pl.ds(r, 1), :], 0.0))
        cz = jnp.sum(jnp.where(sel, pts_ref[0, 2, pl.ds(r, 1), :], 0.0))
        m = out_flat == i
        acc_ref[0] = jnp.where(m, cx, acc_ref[0])
        acc_ref[1] = jnp.where(m, cy, acc_ref[1])
        acc_ref[2] = jnp.where(m, cz, acc_ref[2])
        dx = X - cx
        dy = Y - cy
        dz = Z - cz
        d = (dx * dx + dy * dy) + dz * dz
        nd = jnp.minimum(dist_ref[...], d)
        dist_ref[...] = nd
        mx = jnp.max(nd)
        return jnp.min(jnp.where(nd == mx, flat, N))

    jax.lax.fori_loop(0, NPOINT, body, far0_ref[0, 0, 0], unroll=False)
    ox_ref[0] = acc_ref[0]
    oy_ref[0] = acc_ref[1]
    oz_ref[0] = acc_ref[2]


def _fps_pallas(pts):
    far0 = jax.random.randint(jax.random.key(42), (B,), 0, N).astype(jnp.int32)
    far0 = far0.reshape(B, 1, 1)
    ox, oy, oz = pl.pallas_call(
        _fps_kernel,
        grid=(B,),
        in_specs=[
            pl.BlockSpec((1, 1, 1), lambda b: (b, 0, 0)),
            pl.BlockSpec((1, 3, NR, NL), lambda b: (b, 0, 0, 0)),
        ],
        out_specs=[pl.BlockSpec((1, 4, 128), lambda b: (b, 0, 0))] * 3,
        out_shape=[jax.ShapeDtypeStruct((B, 4, 128), jnp.float32)] * 3,
        scratch_shapes=[
            pltpu.VMEM((NR, NL), jnp.float32),
            pltpu.VMEM((3, 4, 128), jnp.float32),
        ],
        compiler_params=pltpu.CompilerParams(
            dimension_semantics=("arbitrary",),
        ),
    )(far0, pts)
    return jnp.stack([ox, oy, oz], axis=-1).reshape(B, NPOINT, 3)


CU = 4
NREG = 16
BIG = 1 << 20


def _bq_kernel(pts_ref, cx_ref, cy_ref, cz_ref, oval_ref, oidx_ref, d_scr):
    lane16 = jax.lax.broadcasted_iota(jnp.int32, (1, NREG), 1)
    lane32 = jax.lax.broadcasted_iota(jnp.int32, (1, K), 1)
    flatblk = (jax.lax.broadcasted_iota(jnp.int32, (8, NL), 0) * NL
               + jax.lax.broadcasted_iota(jnp.int32, (8, NL), 1))
    inf = jnp.float32(jnp.inf)

    def body(cg, carry):
        Rs = []
        for u in range(CU):
            c = cg * CU + u
            cxv = cx_ref[0, pl.ds(c, 1), :]
            cyv = cy_ref[0, pl.ds(c, 1), :]
            czv = cz_ref[0, pl.ds(c, 1), :]
            dx = pts_ref[0, 0] - cxv
            dy = pts_ref[0, 1] - cyv
            dz = pts_ref[0, 2] - czv
            d = (dx * dx + dy * dy) + dz * dz
            d = jnp.where(d <= THR2, d, inf)
            d_scr[u] = d
            R = jnp.full((1, NREG), inf, jnp.float32)
            for v in range(NREG):
                mv = jnp.min(d[v * 8:(v + 1) * 8, :], axis=(0, 1), keepdims=True)
                R = jnp.where(lane16 == v, mv, R)
            Rs.append(R)

        accs = [(jnp.zeros((1, K), jnp.float32), jnp.zeros((1, K), jnp.int32))
                for _ in range(CU)]
        for k in range(K):
            sk = lane32 == k
            for u in range(CU):
                R = Rs[u]
                acc_val, acc_idx = accs[u]
                g = jnp.min(R, axis=(0, 1), keepdims=True)
                v_vec = jnp.min(jnp.where(R == g, lane16, NREG), axis=(0, 1),
                                keepdims=True)
                v = v_vec[0, 0]
                blk = d_scr[u, pl.ds(v * 8, 8), :]
                rel = jnp.min(jnp.where(blk == g, flatblk, BIG), axis=(0, 1),
                              keepdims=True)
                fid = rel + v_vec * (8 * NL)
                acc_val = jnp.where(sk, g, acc_val)
                acc_idx = jnp.where(sk, fid, acc_idx)
                blk = jnp.where(flatblk == rel, inf, blk)
                d_scr[u, pl.ds(v * 8, 8), :] = blk
                nm = jnp.min(blk, axis=(0, 1), keepdims=True)
                Rs[u] = jnp.where(lane16 == v_vec, nm, R)
                accs[u] = (acc_val, acc_idx)
        for u in range(CU):
            c = cg * CU + u
            oval_ref[0, pl.ds(c, 1), :] = accs[u][0]
            oidx_ref[0, pl.ds(c, 1), :] = accs[u][1]
        return carry

    jax.lax.fori_loop(0, QB // CU, body, 0, unroll=False)


def _ball_query_pallas(pts, new_xyz):
    cx = new_xyz[:, :, 0:1]
    cy = new_xyz[:, :, 1:2]
    cz = new_xyz[:, :, 2:3]
    vals, idx = pl.pallas_call(
        _bq_kernel,
        grid=(B, NPOINT // QB),
        in_specs=[
            pl.BlockSpec((1, 3, NR, NL), lambda b, q: (b, 0, 0, 0)),
            pl.BlockSpec((1, QB, 1), lambda b, q: (b, q, 0)),
            pl.BlockSpec((1, QB, 1), lambda b, q: (b, q, 0)),
            pl.BlockSpec((1, QB, 1), lambda b, q: (b, q, 0)),
        ],
        out_specs=[
            pl.BlockSpec((1, QB, K), lambda b, q: (b, q, 0)),
            pl.BlockSpec((1, QB, K), lambda b, q: (b, q, 0)),
        ],
        out_shape=[
            jax.ShapeDtypeStruct((B, NPOINT, K), jnp.float32),
            jax.ShapeDtypeStruct((B, NPOINT, K), jnp.int32),
        ],
        scratch_shapes=[pltpu.VMEM((CU, NR, NL), jnp.float32)],
        compiler_params=pltpu.CompilerParams(
            dimension_semantics=("parallel", "parallel"),
        ),
    )(pts, cx, cy, cz)
    idx32 = jnp.where(jnp.isinf(vals), -1, idx)
    n04 = jnp.sum((vals <= THR1).astype(jnp.int32), axis=-1, keepdims=True)
    s16 = jnp.arange(16, dtype=jnp.int32)[None, None, :]
    idx16 = jnp.where(s16 < n04, idx[:, :, :16], -1)
    return idx16, idx32


def _bn_relu(x, gamma, beta, eps=1e-5):
    mean = jnp.mean(x, axis=(0, 2, 3), keepdims=True)
    var = jnp.mean((x - mean) ** 2, axis=(0, 2, 3), keepdims=True)
    y = (x - mean) / jnp.sqrt(var + eps)
    y = y * gamma[None, :, None, None] + beta[None, :, None, None]
    return jax.nn.relu(y)


def _forward_core(xyz, features, params, new_xyz, idxs):
    feat_NC = jnp.transpose(features, (0, 2, 1))
    outs = []
    for i, nsample in enumerate(NSAMPLES):
        idx = idxs[i]
        idx_c = jnp.clip(idx, 0, None)
        grouped_xyz = jnp.take_along_axis(xyz[:, None, :, :], idx_c[:, :, :, None], axis=2)
        grouped_xyz = grouped_xyz - new_xyz[:, :, None, :]
        invalid = (idx < 0)[..., None]
        grouped_xyz = jnp.where(invalid, 0.0, grouped_xyz)
        grouped_feat = jnp.take_along_axis(feat_NC[:, None, :, :], idx_c[:, :, :, None], axis=2)
        grouped_feat = jnp.where(invalid, 0.0, grouped_feat)
        grouped = jnp.concatenate([grouped_feat, grouped_xyz], axis=-1)
        x = jnp.transpose(grouped, (0, 3, 1, 2))
        for j in range(len(MLPS[i]) - 1):
            W = params['W%d_%d' % (i, j)]
            x = jnp.einsum('oi,biqs->boqs', W, x)
            x = _bn_relu(x, params['gamma%d_%d' % (i, j)], params['beta%d_%d' % (i, j)])
        outs.append(jnp.max(x, axis=-1))
    return jnp.concatenate(outs, axis=1)


def kernel(xyz, features, W0_0, gamma0_0, beta0_0, W0_1, gamma0_1, beta0_1,
           W1_0, gamma1_0, beta1_0, W1_1, gamma1_1, beta1_1):
    params = {
        'W0_0': W0_0, 'gamma0_0': gamma0_0, 'beta0_0': beta0_0,
        'W0_1': W0_1, 'gamma0_1': gamma0_1, 'beta0_1': beta0_1,
        'W1_0': W1_0, 'gamma1_0': gamma1_0, 'beta1_0': beta1_0,
        'W1_1': W1_1, 'gamma1_1': gamma1_1, 'beta1_1': beta1_1,
    }
    pts = jnp.transpose(xyz, (0, 2, 1)).reshape(B, 3, NR, NL)
    new_xyz = _fps_pallas(pts)
    idx16, idx32 = _ball_query_pallas(pts, new_xyz)
    new_features = _forward_core(xyz, features, params, new_xyz, [idx16, idx32])
    return (new_xyz, new_features)


# speedup vs baseline: 4.3313x; 4.3313x over previous
import jax
import jax.numpy as jnp
import numpy as np
from jax.experimental import pallas as pl
from jax.experimental.pallas import tpu as pltpu

B = 4
N = 16384
C_FEAT = 16
NPOINT = 512
NR, NL = 128, 128
QB = 128
K = 32
THR1 = np.float32(0.2 * 0.2)
THR2 = np.float32(0.4 * 0.4)
MLPS = [[19, 32, 32], [19, 32, 64]]
NSAMPLES = [16, 32]


def _fps_kernel(far0_ref, pts_ref, ox_ref, oy_ref, oz_ref, dist_ref, acc_ref):
    X = pts_ref[0, 0]
    Y = pts_ref[0, 1]
    Z = pts_ref[0, 2]
    flat = (jax.lax.broadcasted_iota(jnp.int32, (NR, NL), 0) * NL
            + jax.lax.broadcasted_iota(jnp.int32, (NR, NL), 1))
    lane1 = jax.lax.broadcasted_iota(jnp.int32, (1, NL), 1)
    out_flat = (jax.lax.broadcasted_iota(jnp.int32, (4, 128), 0) * 128
                + jax.lax.broadcasted_iota(jnp.int32, (4, 128), 1))

    dist_ref[...] = jnp.full((NR, NL), jnp.inf, jnp.float32)
    acc_ref[...] = jnp.zeros((3, 4, 128), jnp.float32)

    def body(i, far):
        r = far // NL
        l = far % NL
        sel = lane1 == l
        cx = jnp.sum(jnp.where(sel, pts_ref[0, 0, pl.ds(r, 1), :], 0.0))
        cy = jnp.sum(jnp.where(sel, pts_ref[0, 1, pl.ds(r, 1), :], 0.0))
        cz = jnp.sum(jnp.where(sel, pts_ref[0, 2, pl.ds(r, 1), :], 0.0))
        m = out_flat == i
        acc_ref[0] = jnp.where(m, cx, acc_ref[0])
        acc_ref[1] = jnp.where(m, cy, acc_ref[1])
        acc_ref[2] = jnp.where(m, cz, acc_ref[2])
        dx = X - cx
        dy = Y - cy
        dz = Z - cz
        d = (dx * dx + dy * dy) + dz * dz
        nd = jnp.minimum(dist_ref[...], d)
        dist_ref[...] = nd
        mx = jnp.max(nd)
        return jnp.min(jnp.where(nd == mx, flat, N))

    jax.lax.fori_loop(0, NPOINT, body, far0_ref[0, 0, 0], unroll=False)
    ox_ref[0] = acc_ref[0]
    oy_ref[0] = acc_ref[1]
    oz_ref[0] = acc_ref[2]


def _fps_pallas(pts):
    far0 = jax.random.randint(jax.random.key(42), (B,), 0, N).astype(jnp.int32)
    far0 = far0.reshape(B, 1, 1)
    ox, oy, oz = pl.pallas_call(
        _fps_kernel,
        grid=(B,),
        in_specs=[
            pl.BlockSpec((1, 1, 1), lambda b: (b, 0, 0)),
            pl.BlockSpec((1, 3, NR, NL), lambda b: (b, 0, 0, 0)),
        ],
        out_specs=[pl.BlockSpec((1, 4, 128), lambda b: (b, 0, 0))] * 3,
        out_shape=[jax.ShapeDtypeStruct((B, 4, 128), jnp.float32)] * 3,
        scratch_shapes=[
            pltpu.VMEM((NR, NL), jnp.float32),
            pltpu.VMEM((3, 4, 128), jnp.float32),
        ],
        compiler_params=pltpu.CompilerParams(
            dimension_semantics=("arbitrary",),
        ),
    )(far0, pts)
    return jnp.stack([ox, oy, oz], axis=-1).reshape(B, NPOINT, 3)


CU = 8
NREG = 16
BIG = 1 << 20


def _bq_kernel(pts_ref, cx_ref, cy_ref, cz_ref, oval_ref, oidx_ref, *d_scrs):
    lane16 = jax.lax.broadcasted_iota(jnp.int32, (1, NREG), 1)
    lane32 = jax.lax.broadcasted_iota(jnp.int32, (1, K), 1)
    flatblk = (jax.lax.broadcasted_iota(jnp.int32, (8, NL), 0) * NL
               + jax.lax.broadcasted_iota(jnp.int32, (8, NL), 1))
    inf = jnp.float32(jnp.inf)

    def body(cg, carry):
        Rs = []
        for u in range(CU):
            c = cg * CU + u
            cxv = cx_ref[0, pl.ds(c, 1), :]
            cyv = cy_ref[0, pl.ds(c, 1), :]
            czv = cz_ref[0, pl.ds(c, 1), :]
            dx = pts_ref[0, 0] - cxv
            dy = pts_ref[0, 1] - cyv
            dz = pts_ref[0, 2] - czv
            d = (dx * dx + dy * dy) + dz * dz
            d = jnp.where(d <= THR2, d, inf)
            d_scrs[u][...] = d
            R = jnp.full((1, NREG), inf, jnp.float32)
            for v in range(NREG):
                mv = jnp.min(d[v * 8:(v + 1) * 8, :], axis=(0, 1), keepdims=True)
                R = jnp.where(lane16 == v, mv, R)
            Rs.append(R)

        accs = [(jnp.zeros((1, K), jnp.float32), jnp.zeros((1, K), jnp.int32))
                for _ in range(CU)]
        for k in range(K):
            sk = lane32 == k
            gs = [jnp.min(Rs[u], axis=(0, 1), keepdims=True) for u in range(CU)]
            vvecs = [jnp.min(jnp.where(Rs[u] == gs[u], lane16, NREG),
                             axis=(0, 1), keepdims=True) for u in range(CU)]
            vss = [vvecs[u][0, 0] for u in range(CU)]
            blks = [d_scrs[u][pl.ds(vss[u] * 8, 8), :] for u in range(CU)]
            rels = [jnp.min(jnp.where(blks[u] == gs[u], flatblk, BIG),
                            axis=(0, 1), keepdims=True) for u in range(CU)]
            fids = [rels[u] + vvecs[u] * (8 * NL) for u in range(CU)]
            blks = [jnp.where(flatblk == rels[u], inf, blks[u])
                    for u in range(CU)]
            for u in range(CU):
                d_scrs[u][pl.ds(vss[u] * 8, 8), :] = blks[u]
            nms = [jnp.min(blks[u], axis=(0, 1), keepdims=True)
                   for u in range(CU)]
            for u in range(CU):
                acc_val, acc_idx = accs[u]
                accs[u] = (jnp.where(sk, gs[u], acc_val),
                           jnp.where(sk, fids[u], acc_idx))
                Rs[u] = jnp.where(lane16 == vvecs[u], nms[u], Rs[u])
        for u in range(CU):
            c = cg * CU + u
            oval_ref[0, pl.ds(c, 1), :] = accs[u][0]
            oidx_ref[0, pl.ds(c, 1), :] = accs[u][1]
        return carry

    jax.lax.fori_loop(0, QB // CU, body, 0, unroll=False)


def _ball_query_pallas(pts, new_xyz):
    cx = new_xyz[:, :, 0:1]
    cy = new_xyz[:, :, 1:2]
    cz = new_xyz[:, :, 2:3]
    vals, idx = pl.pallas_call(
        _bq_kernel,
        grid=(B, NPOINT // QB),
        in_specs=[
            pl.BlockSpec((1, 3, NR, NL), lambda b, q: (b, 0, 0, 0)),
            pl.BlockSpec((1, QB, 1), lambda b, q: (b, q, 0)),
            pl.BlockSpec((1, QB, 1), lambda b, q: (b, q, 0)),
            pl.BlockSpec((1, QB, 1), lambda b, q: (b, q, 0)),
        ],
        out_specs=[
            pl.BlockSpec((1, QB, K), lambda b, q: (b, q, 0)),
            pl.BlockSpec((1, QB, K), lambda b, q: (b, q, 0)),
        ],
        out_shape=[
            jax.ShapeDtypeStruct((B, NPOINT, K), jnp.float32),
            jax.ShapeDtypeStruct((B, NPOINT, K), jnp.int32),
        ],
        scratch_shapes=[pltpu.VMEM((NR, NL), jnp.float32)] * CU,
        compiler_params=pltpu.CompilerParams(
            dimension_semantics=("parallel", "parallel"),
        ),
    )(pts, cx, cy, cz)
    idx32 = jnp.where(jnp.isinf(vals), -1, idx)
    n04 = jnp.sum((vals <= THR1).astype(jnp.int32), axis=-1, keepdims=True)
    s16 = jnp.arange(16, dtype=jnp.int32)[None, None, :]
    idx16 = jnp.where(s16 < n04, idx[:, :, :16], -1)
    return idx16, idx32


def _bn_relu(x, gamma, beta, eps=1e-5):
    mean = jnp.mean(x, axis=(0, 2, 3), keepdims=True)
    var = jnp.mean((x - mean) ** 2, axis=(0, 2, 3), keepdims=True)
    y = (x - mean) / jnp.sqrt(var + eps)
    y = y * gamma[None, :, None, None] + beta[None, :, None, None]
    return jax.nn.relu(y)


def _forward_core(xyz, features, params, new_xyz, idxs):
    feat_NC = jnp.transpose(features, (0, 2, 1))
    outs = []
    for i, nsample in enumerate(NSAMPLES):
        idx = idxs[i]
        idx_c = jnp.clip(idx, 0, None)
        grouped_xyz = jnp.take_along_axis(xyz[:, None, :, :], idx_c[:, :, :, None], axis=2)
        grouped_xyz = grouped_xyz - new_xyz[:, :, None, :]
        invalid = (idx < 0)[..., None]
        grouped_xyz = jnp.where(invalid, 0.0, grouped_xyz)
        grouped_feat = jnp.take_along_axis(feat_NC[:, None, :, :], idx_c[:, :, :, None], axis=2)
        grouped_feat = jnp.where(invalid, 0.0, grouped_feat)
        grouped = jnp.concatenate([grouped_feat, grouped_xyz], axis=-1)
        x = jnp.transpose(grouped, (0, 3, 1, 2))
        for j in range(len(MLPS[i]) - 1):
            W = params['W%d_%d' % (i, j)]
            x = jnp.einsum('oi,biqs->boqs', W, x)
            x = _bn_relu(x, params['gamma%d_%d' % (i, j)], params['beta%d_%d' % (i, j)])
        outs.append(jnp.max(x, axis=-1))
    return jnp.concatenate(outs, axis=1)


def kernel(xyz, features, W0_0, gamma0_0, beta0_0, W0_1, gamma0_1, beta0_1,
           W1_0, gamma1_0, beta1_0, W1_1, gamma1_1, beta1_1):
    params = {
        'W0_0': W0_0, 'gamma0_0': gamma0_0, 'beta0_0': beta0_0,
        'W0_1': W0_1, 'gamma0_1': gamma0_1, 'beta0_1': beta0_1,
        'W1_0': W1_0, 'gamma1_0': gamma1_0, 'beta1_0': beta1_0,
        'W1_1': W1_1, 'gamma1_1': gamma1_1, 'beta1_1': beta1_1,
    }
    pts = jnp.transpose(xyz, (0, 2, 1)).reshape(B, 3, NR, NL)
    new_xyz = _fps_pallas(pts)
    idx16, idx32 = _ball_query_pallas(pts, new_xyz)
    new_features = _forward_core(xyz, features, params, new_xyz, [idx16, idx32])
    return (new_xyz, new_features)


# speedup vs baseline: 6.1611x; 1.4224x over previous
import jax
import jax.numpy as jnp
import numpy as np
from jax.experimental import pallas as pl
from jax.experimental.pallas import tpu as pltpu

B = 4
N = 16384
C_FEAT = 16
NPOINT = 512
NR, NL = 128, 128
QB = 128
K = 32
THR1 = np.float32(0.2 * 0.2)
THR2 = np.float32(0.4 * 0.4)
MLPS = [[19, 32, 32], [19, 32, 64]]
NSAMPLES = [16, 32]


def _fps_kernel(far0_ref, pts_ref, ox_ref, oy_ref, oz_ref, dist_ref, acc_ref):
    X = pts_ref[0, 0]
    Y = pts_ref[0, 1]
    Z = pts_ref[0, 2]
    flat = (jax.lax.broadcasted_iota(jnp.int32, (NR, NL), 0) * NL
            + jax.lax.broadcasted_iota(jnp.int32, (NR, NL), 1))
    lane1 = jax.lax.broadcasted_iota(jnp.int32, (1, NL), 1)
    out_flat = (jax.lax.broadcasted_iota(jnp.int32, (4, 128), 0) * 128
                + jax.lax.broadcasted_iota(jnp.int32, (4, 128), 1))

    dist_ref[...] = jnp.full((NR, NL), jnp.inf, jnp.float32)
    acc_ref[...] = jnp.zeros((3, 4, 128), jnp.float32)

    def body(i, far):
        r = far // NL
        l = far % NL
        sel = lane1 == l
        cx = jnp.sum(jnp.where(sel, pts_ref[0, 0, pl.ds(r, 1), :], 0.0))
        cy = jnp.sum(jnp.where(sel, pts_ref[0, 1, pl.ds(r, 1), :], 0.0))
        cz = jnp.sum(jnp.where(sel, pts_ref[0, 2, pl.ds(r, 1), :], 0.0))
        m = out_flat == i
        acc_ref[0] = jnp.where(m, cx, acc_ref[0])
        acc_ref[1] = jnp.where(m, cy, acc_ref[1])
        acc_ref[2] = jnp.where(m, cz, acc_ref[2])
        dx = X - cx
        dy = Y - cy
        dz = Z - cz
        d = (dx * dx + dy * dy) + dz * dz
        nd = jnp.minimum(dist_ref[...], d)
        dist_ref[...] = nd
        mx = jnp.max(nd)
        return jnp.min(jnp.where(nd == mx, flat, N))

    jax.lax.fori_loop(0, NPOINT, body, far0_ref[0, 0, 0], unroll=False)
    ox_ref[0] = acc_ref[0]
    oy_ref[0] = acc_ref[1]
    oz_ref[0] = acc_ref[2]


def _fps_pallas(pts):
    far0 = jax.random.randint(jax.random.key(42), (B,), 0, N).astype(jnp.int32)
    far0 = far0.reshape(B, 1, 1)
    ox, oy, oz = pl.pallas_call(
        _fps_kernel,
        grid=(B,),
        in_specs=[
            pl.BlockSpec((1, 1, 1), lambda b: (b, 0, 0)),
            pl.BlockSpec((1, 3, NR, NL), lambda b: (b, 0, 0, 0)),
        ],
        out_specs=[pl.BlockSpec((1, 4, 128), lambda b: (b, 0, 0))] * 3,
        out_shape=[jax.ShapeDtypeStruct((B, 4, 128), jnp.float32)] * 3,
        scratch_shapes=[
            pltpu.VMEM((NR, NL), jnp.float32),
            pltpu.VMEM((3, 4, 128), jnp.float32),
        ],
        compiler_params=pltpu.CompilerParams(
            dimension_semantics=("arbitrary",),
        ),
    )(far0, pts)
    return jnp.stack([ox, oy, oz], axis=-1).reshape(B, NPOINT, 3)


CU = 16
NREG = 16
BIG = 1 << 20


def _bq_kernel(pts_ref, cx_ref, cy_ref, cz_ref, oval_ref, oidx_ref, *d_scrs):
    lane16 = jax.lax.broadcasted_iota(jnp.int32, (1, NREG), 1)
    lane32 = jax.lax.broadcasted_iota(jnp.int32, (1, K), 1)
    flatblk = (jax.lax.broadcasted_iota(jnp.int32, (8, NL), 0) * NL
               + jax.lax.broadcasted_iota(jnp.int32, (8, NL), 1))
    inf = jnp.float32(jnp.inf)

    def body(cg, carry):
        Rs = []
        for u in range(CU):
            c = cg * CU + u
            cxv = cx_ref[0, pl.ds(c, 1), :]
            cyv = cy_ref[0, pl.ds(c, 1), :]
            czv = cz_ref[0, pl.ds(c, 1), :]
            dx = pts_ref[0, 0] - cxv
            dy = pts_ref[0, 1] - cyv
            dz = pts_ref[0, 2] - czv
            d = (dx * dx + dy * dy) + dz * dz
            d = jnp.where(d <= THR2, d, inf)
            d_scrs[u][...] = d
            R = jnp.full((1, NREG), inf, jnp.float32)
            for v in range(NREG):
                mv = jnp.min(d[v * 8:(v + 1) * 8, :], axis=(0, 1), keepdims=True)
                R = jnp.where(lane16 == v, mv, R)
            Rs.append(R)

        accs = [(jnp.zeros((1, K), jnp.float32), jnp.zeros((1, K), jnp.int32))
                for _ in range(CU)]
        for k in range(K):
            sk = lane32 == k
            gs = [jnp.min(Rs[u], axis=(0, 1), keepdims=True) for u in range(CU)]
            vvecs = [jnp.min(jnp.where(Rs[u] == gs[u], lane16, NREG),
                             axis=(0, 1), keepdims=True) for u in range(CU)]
            vss = [vvecs[u][0, 0] for u in range(CU)]
            blks = [d_scrs[u][pl.ds(vss[u] * 8, 8), :] for u in range(CU)]
            rels = [jnp.min(jnp.where(blks[u] == gs[u], flatblk, BIG),
                            axis=(0, 1), keepdims=True) for u in range(CU)]
            fids = [rels[u] + vvecs[u] * (8 * NL) for u in range(CU)]
            blks = [jnp.where(flatblk == rels[u], inf, blks[u])
                    for u in range(CU)]
            for u in range(CU):
                d_scrs[u][pl.ds(vss[u] * 8, 8), :] = blks[u]
            nms = [jnp.min(blks[u], axis=(0, 1), keepdims=True)
                   for u in range(CU)]
            for u in range(CU):
                acc_val, acc_idx = accs[u]
                accs[u] = (jnp.where(sk, gs[u], acc_val),
                           jnp.where(sk, fids[u], acc_idx))
                Rs[u] = jnp.where(lane16 == vvecs[u], nms[u], Rs[u])
        for u in range(CU):
            c = cg * CU + u
            oval_ref[0, pl.ds(c, 1), :] = accs[u][0]
            oidx_ref[0, pl.ds(c, 1), :] = accs[u][1]
        return carry

    jax.lax.fori_loop(0, QB // CU, body, 0, unroll=False)


def _ball_query_pallas(pts, new_xyz):
    cx = new_xyz[:, :, 0:1]
    cy = new_xyz[:, :, 1:2]
    cz = new_xyz[:, :, 2:3]
    vals, idx = pl.pallas_call(
        _bq_kernel,
        grid=(B, NPOINT // QB),
        in_specs=[
            pl.BlockSpec((1, 3, NR, NL), lambda b, q: (b, 0, 0, 0)),
            pl.BlockSpec((1, QB, 1), lambda b, q: (b, q, 0)),
            pl.BlockSpec((1, QB, 1), lambda b, q: (b, q, 0)),
            pl.BlockSpec((1, QB, 1), lambda b, q: (b, q, 0)),
        ],
        out_specs=[
            pl.BlockSpec((1, QB, K), lambda b, q: (b, q, 0)),
            pl.BlockSpec((1, QB, K), lambda b, q: (b, q, 0)),
        ],
        out_shape=[
            jax.ShapeDtypeStruct((B, NPOINT, K), jnp.float32),
            jax.ShapeDtypeStruct((B, NPOINT, K), jnp.int32),
        ],
        scratch_shapes=[pltpu.VMEM((NR, NL), jnp.float32)] * CU,
        compiler_params=pltpu.CompilerParams(
            dimension_semantics=("parallel", "parallel"),
        ),
    )(pts, cx, cy, cz)
    idx32 = jnp.where(jnp.isinf(vals), -1, idx)
    n04 = jnp.sum((vals <= THR1).astype(jnp.int32), axis=-1, keepdims=True)
    s16 = jnp.arange(16, dtype=jnp.int32)[None, None, :]
    idx16 = jnp.where(s16 < n04, idx[:, :, :16], -1)
    return idx16, idx32


def _bn_relu(x, gamma, beta, eps=1e-5):
    mean = jnp.mean(x, axis=(0, 2, 3), keepdims=True)
    var = jnp.mean((x - mean) ** 2, axis=(0, 2, 3), keepdims=True)
    y = (x - mean) / jnp.sqrt(var + eps)
    y = y * gamma[None, :, None, None] + beta[None, :, None, None]
    return jax.nn.relu(y)


def _forward_core(xyz, features, params, new_xyz, idxs):
    feat_NC = jnp.transpose(features, (0, 2, 1))
    outs = []
    for i, nsample in enumerate(NSAMPLES):
        idx = idxs[i]
        idx_c = jnp.clip(idx, 0, None)
        grouped_xyz = jnp.take_along_axis(xyz[:, None, :, :], idx_c[:, :, :, None], axis=2)
        grouped_xyz = grouped_xyz - new_xyz[:, :, None, :]
        invalid = (idx < 0)[..., None]
        grouped_xyz = jnp.where(invalid, 0.0, grouped_xyz)
        grouped_feat = jnp.take_along_axis(feat_NC[:, None, :, :], idx_c[:, :, :, None], axis=2)
        grouped_feat = jnp.where(invalid, 0.0, grouped_feat)
        grouped = jnp.concatenate([grouped_feat, grouped_xyz], axis=-1)
        x = jnp.transpose(grouped, (0, 3, 1, 2))
        for j in range(len(MLPS[i]) - 1):
            W = params['W%d_%d' % (i, j)]
            x = jnp.einsum('oi,biqs->boqs', W, x)
            x = _bn_relu(x, params['gamma%d_%d' % (i, j)], params['beta%d_%d' % (i, j)])
        outs.append(jnp.max(x, axis=-1))
    return jnp.concatenate(outs, axis=1)


def kernel(xyz, features, W0_0, gamma0_0, beta0_0, W0_1, gamma0_1, beta0_1,
           W1_0, gamma1_0, beta1_0, W1_1, gamma1_1, beta1_1):
    params = {
        'W0_0': W0_0, 'gamma0_0': gamma0_0, 'beta0_0': beta0_0,
        'W0_1': W0_1, 'gamma0_1': gamma0_1, 'beta0_1': beta0_1,
        'W1_0': W1_0, 'gamma1_0': gamma1_0, 'beta1_0': beta1_0,
        'W1_1': W1_1, 'gamma1_1': gamma1_1, 'beta1_1': beta1_1,
    }
    pts = jnp.transpose(xyz, (0, 2, 1)).reshape(B, 3, NR, NL)
    new_xyz = _fps_pallas(pts)
    idx16, idx32 = _ball_query_pallas(pts, new_xyz)
    new_features = _forward_core(xyz, features, params, new_xyz, [idx16, idx32])
    return (new_xyz, new_features)


# speedup vs baseline: 7.7112x; 1.2516x over previous
import jax
import jax.numpy as jnp
import numpy as np
from jax.experimental import pallas as pl
from jax.experimental.pallas import tpu as pltpu

B = 4
N = 16384
C_FEAT = 16
NPOINT = 512
NR, NL = 128, 128
QB = 128
K = 32
THR1 = np.float32(0.2 * 0.2)
THR2 = np.float32(0.4 * 0.4)
MLPS = [[19, 32, 32], [19, 32, 64]]
NSAMPLES = [16, 32]


def _fps_kernel(far0_ref, pts_ref, ox_ref, oy_ref, oz_ref, dist_ref, acc_ref):
    X = pts_ref[0, 0]
    Y = pts_ref[0, 1]
    Z = pts_ref[0, 2]
    flat = (jax.lax.broadcasted_iota(jnp.int32, (NR, NL), 0) * NL
            + jax.lax.broadcasted_iota(jnp.int32, (NR, NL), 1))
    lane1 = jax.lax.broadcasted_iota(jnp.int32, (1, NL), 1)
    out_flat = (jax.lax.broadcasted_iota(jnp.int32, (4, 128), 0) * 128
                + jax.lax.broadcasted_iota(jnp.int32, (4, 128), 1))

    dist_ref[...] = jnp.full((NR, NL), jnp.inf, jnp.float32)
    acc_ref[...] = jnp.zeros((3, 4, 128), jnp.float32)

    def body(i, far):
        r = far // NL
        l = far % NL
        sel = lane1 == l
        cx = jnp.sum(jnp.where(sel, pts_ref[0, 0, pl.ds(r, 1), :], 0.0))
        cy = jnp.sum(jnp.where(sel, pts_ref[0, 1, pl.ds(r, 1), :], 0.0))
        cz = jnp.sum(jnp.where(sel, pts_ref[0, 2, pl.ds(r, 1), :], 0.0))
        m = out_flat == i
        acc_ref[0] = jnp.where(m, cx, acc_ref[0])
        acc_ref[1] = jnp.where(m, cy, acc_ref[1])
        acc_ref[2] = jnp.where(m, cz, acc_ref[2])
        dx = X - cx
        dy = Y - cy
        dz = Z - cz
        d = (dx * dx + dy * dy) + dz * dz
        nd = jnp.minimum(dist_ref[...], d)
        dist_ref[...] = nd
        mx = jnp.max(nd)
        return jnp.min(jnp.where(nd == mx, flat, N))

    jax.lax.fori_loop(0, NPOINT, body, far0_ref[0, 0, 0], unroll=False)
    ox_ref[0] = acc_ref[0]
    oy_ref[0] = acc_ref[1]
    oz_ref[0] = acc_ref[2]


def _fps_pallas(pts):
    far0 = jax.random.randint(jax.random.key(42), (B,), 0, N).astype(jnp.int32)
    far0 = far0.reshape(B, 1, 1)
    ox, oy, oz = pl.pallas_call(
        _fps_kernel,
        grid=(B,),
        in_specs=[
            pl.BlockSpec((1, 1, 1), lambda b: (b, 0, 0)),
            pl.BlockSpec((1, 3, NR, NL), lambda b: (b, 0, 0, 0)),
        ],
        out_specs=[pl.BlockSpec((1, 4, 128), lambda b: (b, 0, 0))] * 3,
        out_shape=[jax.ShapeDtypeStruct((B, 4, 128), jnp.float32)] * 3,
        scratch_shapes=[
            pltpu.VMEM((NR, NL), jnp.float32),
            pltpu.VMEM((3, 4, 128), jnp.float32),
        ],
        compiler_params=pltpu.CompilerParams(
            dimension_semantics=("arbitrary",),
        ),
    )(far0, pts)
    return jnp.stack([ox, oy, oz], axis=-1).reshape(B, NPOINT, 3)


CU = 32
NREG = 16
BIG = 1 << 20


def _bq_kernel(pts_ref, cx_ref, cy_ref, cz_ref, oval_ref, oidx_ref, *d_scrs):
    lane16 = jax.lax.broadcasted_iota(jnp.int32, (1, NREG), 1)
    lane32 = jax.lax.broadcasted_iota(jnp.int32, (1, K), 1)
    flatblk = (jax.lax.broadcasted_iota(jnp.int32, (8, NL), 0) * NL
               + jax.lax.broadcasted_iota(jnp.int32, (8, NL), 1))
    inf = jnp.float32(jnp.inf)

    def body(cg, carry):
        Rs = []
        for u in range(CU):
            c = cg * CU + u
            cxv = cx_ref[0, pl.ds(c, 1), :]
            cyv = cy_ref[0, pl.ds(c, 1), :]
            czv = cz_ref[0, pl.ds(c, 1), :]
            dx = pts_ref[0, 0] - cxv
            dy = pts_ref[0, 1] - cyv
            dz = pts_ref[0, 2] - czv
            d = (dx * dx + dy * dy) + dz * dz
            d = jnp.where(d <= THR2, d, inf)
            d_scrs[u][...] = d
            R = jnp.full((1, NREG), inf, jnp.float32)
            for v in range(NREG):
                mv = jnp.min(d[v * 8:(v + 1) * 8, :], axis=(0, 1), keepdims=True)
                R = jnp.where(lane16 == v, mv, R)
            Rs.append(R)

        accs = [(jnp.zeros((1, K), jnp.float32), jnp.zeros((1, K), jnp.int32))
                for _ in range(CU)]
        for k in range(K):
            sk = lane32 == k
            gs = [jnp.min(Rs[u], axis=(0, 1), keepdims=True) for u in range(CU)]
            vvecs = [jnp.min(jnp.where(Rs[u] == gs[u], lane16, NREG),
                             axis=(0, 1), keepdims=True) for u in range(CU)]
            vss = [vvecs[u][0, 0] for u in range(CU)]
            blks = [d_scrs[u][pl.ds(vss[u] * 8, 8), :] for u in range(CU)]
            rels = [jnp.min(jnp.where(blks[u] == gs[u], flatblk, BIG),
                            axis=(0, 1), keepdims=True) for u in range(CU)]
            fids = [rels[u] + vvecs[u] * (8 * NL) for u in range(CU)]
            blks = [jnp.where(flatblk == rels[u], inf, blks[u])
                    for u in range(CU)]
            for u in range(CU):
                d_scrs[u][pl.ds(vss[u] * 8, 8), :] = blks[u]
            nms = [jnp.min(blks[u], axis=(0, 1), keepdims=True)
                   for u in range(CU)]
            for u in range(CU):
                acc_val, acc_idx = accs[u]
                accs[u] = (jnp.where(sk, gs[u], acc_val),
                           jnp.where(sk, fids[u], acc_idx))
                Rs[u] = jnp.where(lane16 == vvecs[u], nms[u], Rs[u])
        for u in range(CU):
            c = cg * CU + u
            oval_ref[0, pl.ds(c, 1), :] = accs[u][0]
            oidx_ref[0, pl.ds(c, 1), :] = accs[u][1]
        return carry

    jax.lax.fori_loop(0, QB // CU, body, 0, unroll=False)


def _ball_query_pallas(pts, new_xyz):
    cx = new_xyz[:, :, 0:1]
    cy = new_xyz[:, :, 1:2]
    cz = new_xyz[:, :, 2:3]
    vals, idx = pl.pallas_call(
        _bq_kernel,
        grid=(B, NPOINT // QB),
        in_specs=[
            pl.BlockSpec((1, 3, NR, NL), lambda b, q: (b, 0, 0, 0)),
            pl.BlockSpec((1, QB, 1), lambda b, q: (b, q, 0)),
            pl.BlockSpec((1, QB, 1), lambda b, q: (b, q, 0)),
            pl.BlockSpec((1, QB, 1), lambda b, q: (b, q, 0)),
        ],
        out_specs=[
            pl.BlockSpec((1, QB, K), lambda b, q: (b, q, 0)),
            pl.BlockSpec((1, QB, K), lambda b, q: (b, q, 0)),
        ],
        out_shape=[
            jax.ShapeDtypeStruct((B, NPOINT, K), jnp.float32),
            jax.ShapeDtypeStruct((B, NPOINT, K), jnp.int32),
        ],
        scratch_shapes=[pltpu.VMEM((NR, NL), jnp.float32)] * CU,
        compiler_params=pltpu.CompilerParams(
            dimension_semantics=("parallel", "parallel"),
        ),
    )(pts, cx, cy, cz)
    idx32 = jnp.where(jnp.isinf(vals), -1, idx)
    n04 = jnp.sum((vals <= THR1).astype(jnp.int32), axis=-1, keepdims=True)
    s16 = jnp.arange(16, dtype=jnp.int32)[None, None, :]
    idx16 = jnp.where(s16 < n04, idx[:, :, :16], -1)
    return idx16, idx32


def _bn_relu(x, gamma, beta, eps=1e-5):
    mean = jnp.mean(x, axis=(0, 2, 3), keepdims=True)
    var = jnp.mean((x - mean) ** 2, axis=(0, 2, 3), keepdims=True)
    y = (x - mean) / jnp.sqrt(var + eps)
    y = y * gamma[None, :, None, None] + beta[None, :, None, None]
    return jax.nn.relu(y)


def _forward_core(xyz, features, params, new_xyz, idxs):
    feat_NC = jnp.transpose(features, (0, 2, 1))
    outs = []
    for i, nsample in enumerate(NSAMPLES):
        idx = idxs[i]
        idx_c = jnp.clip(idx, 0, None)
        grouped_xyz = jnp.take_along_axis(xyz[:, None, :, :], idx_c[:, :, :, None], axis=2)
        grouped_xyz = grouped_xyz - new_xyz[:, :, None, :]
        invalid = (idx < 0)[..., None]
        grouped_xyz = jnp.where(invalid, 0.0, grouped_xyz)
        grouped_feat = jnp.take_along_axis(feat_NC[:, None, :, :], idx_c[:, :, :, None], axis=2)
        grouped_feat = jnp.where(invalid, 0.0, grouped_feat)
        grouped = jnp.concatenate([grouped_feat, grouped_xyz], axis=-1)
        x = jnp.transpose(grouped, (0, 3, 1, 2))
        for j in range(len(MLPS[i]) - 1):
            W = params['W%d_%d' % (i, j)]
            x = jnp.einsum('oi,biqs->boqs', W, x)
            x = _bn_relu(x, params['gamma%d_%d' % (i, j)], params['beta%d_%d' % (i, j)])
        outs.append(jnp.max(x, axis=-1))
    return jnp.concatenate(outs, axis=1)


def kernel(xyz, features, W0_0, gamma0_0, beta0_0, W0_1, gamma0_1, beta0_1,
           W1_0, gamma1_0, beta1_0, W1_1, gamma1_1, beta1_1):
    params = {
        'W0_0': W0_0, 'gamma0_0': gamma0_0, 'beta0_0': beta0_0,
        'W0_1': W0_1, 'gamma0_1': gamma0_1, 'beta0_1': beta0_1,
        'W1_0': W1_0, 'gamma1_0': gamma1_0, 'beta1_0': beta1_0,
        'W1_1': W1_1, 'gamma1_1': gamma1_1, 'beta1_1': beta1_1,
    }
    pts = jnp.transpose(xyz, (0, 2, 1)).reshape(B, 3, NR, NL)
    new_xyz = _fps_pallas(pts)
    idx16, idx32 = _ball_query_pallas(pts, new_xyz)
    new_features = _forward_core(xyz, features, params, new_xyz, [idx16, idx32])
    return (new_xyz, new_features)


# speedup vs baseline: 7.7124x; 1.0002x over previous
import jax
import jax.numpy as jnp
import numpy as np
from jax.experimental import pallas as pl
from jax.experimental.pallas import tpu as pltpu

B = 4
N = 16384
C_FEAT = 16
NPOINT = 512
NR, NL = 128, 128
QB = 128
K = 32
THR1 = np.float32(0.2 * 0.2)
THR2 = np.float32(0.4 * 0.4)
MLPS = [[19, 32, 32], [19, 32, 64]]
NSAMPLES = [16, 32]


def _fps_kernel(far0_ref, pts_ref, ox_ref, oy_ref, oz_ref, dist_ref, acc_ref):
    X = pts_ref[0, 0]
    Y = pts_ref[0, 1]
    Z = pts_ref[0, 2]
    flat = (jax.lax.broadcasted_iota(jnp.int32, (NR, NL), 0) * NL
            + jax.lax.broadcasted_iota(jnp.int32, (NR, NL), 1))
    lane1 = jax.lax.broadcasted_iota(jnp.int32, (1, NL), 1)
    out_flat = (jax.lax.broadcasted_iota(jnp.int32, (4, 128), 0) * 128
                + jax.lax.broadcasted_iota(jnp.int32, (4, 128), 1))

    dist_ref[...] = jnp.full((NR, NL), jnp.inf, jnp.float32)
    acc_ref[...] = jnp.zeros((3, 4, 128), jnp.float32)

    def body(i, far):
        r = far // NL
        l = far % NL
        sel = lane1 == l
        cx = jnp.sum(jnp.where(sel, pts_ref[0, 0, pl.ds(r, 1), :], 0.0))
        cy = jnp.sum(jnp.where(sel, pts_ref[0, 1, pl.ds(r, 1), :], 0.0))
        cz = jnp.sum(jnp.where(sel, pts_ref[0, 2, pl.ds(r, 1), :], 0.0))
        m = out_flat == i
        acc_ref[0] = jnp.where(m, cx, acc_ref[0])
        acc_ref[1] = jnp.where(m, cy, acc_ref[1])
        acc_ref[2] = jnp.where(m, cz, acc_ref[2])
        dx = X - cx
        dy = Y - cy
        dz = Z - cz
        d = (dx * dx + dy * dy) + dz * dz
        nd = jnp.minimum(dist_ref[...], d)
        dist_ref[...] = nd
        mx = jnp.max(nd)
        return jnp.min(jnp.where(nd == mx, flat, N))

    jax.lax.fori_loop(0, NPOINT, body, far0_ref[0, 0, 0], unroll=False)
    ox_ref[0] = acc_ref[0]
    oy_ref[0] = acc_ref[1]
    oz_ref[0] = acc_ref[2]


def _fps_pallas(pts):
    far0 = jax.random.randint(jax.random.key(42), (B,), 0, N).astype(jnp.int32)
    far0 = far0.reshape(B, 1, 1)
    ox, oy, oz = pl.pallas_call(
        _fps_kernel,
        grid=(B,),
        in_specs=[
            pl.BlockSpec((1, 1, 1), lambda b: (b, 0, 0)),
            pl.BlockSpec((1, 3, NR, NL), lambda b: (b, 0, 0, 0)),
        ],
        out_specs=[pl.BlockSpec((1, 4, 128), lambda b: (b, 0, 0))] * 3,
        out_shape=[jax.ShapeDtypeStruct((B, 4, 128), jnp.float32)] * 3,
        scratch_shapes=[
            pltpu.VMEM((NR, NL), jnp.float32),
            pltpu.VMEM((3, 4, 128), jnp.float32),
        ],
        compiler_params=pltpu.CompilerParams(
            dimension_semantics=("arbitrary",),
        ),
    )(far0, pts)
    return jnp.stack([ox, oy, oz], axis=-1).reshape(B, NPOINT, 3)


CU = 32
NREG = 16
BIG = 1 << 20


def _bq_kernel(pts_ref, cx_ref, cy_ref, cz_ref, oval_ref, oidx_ref, *d_scrs):
    lane16 = jax.lax.broadcasted_iota(jnp.int32, (1, NREG), 1)
    lane32 = jax.lax.broadcasted_iota(jnp.int32, (1, K), 1)
    flatblk = (jax.lax.broadcasted_iota(jnp.int32, (8, NL), 0) * NL
               + jax.lax.broadcasted_iota(jnp.int32, (8, NL), 1))
    inf = jnp.float32(jnp.inf)

    def body(cg, carry):
        Rs = []
        for u in range(CU):
            c = cg * CU + u
            cxv = cx_ref[0, pl.ds(c, 1), :]
            cyv = cy_ref[0, pl.ds(c, 1), :]
            czv = cz_ref[0, pl.ds(c, 1), :]
            dx = pts_ref[0, 0] - cxv
            dy = pts_ref[0, 1] - cyv
            dz = pts_ref[0, 2] - czv
            d = (dx * dx + dy * dy) + dz * dz
            d = jnp.where(d <= THR2, d, inf)
            d_scrs[u][...] = d
            R = jnp.full((1, NREG), inf, jnp.float32)
            for v in range(NREG):
                mv = jnp.min(d[v * 8:(v + 1) * 8, :], axis=(0, 1), keepdims=True)
                R = jnp.where(lane16 == v, mv, R)
            Rs.append(R)

        accs = [(jnp.zeros((1, K), jnp.float32), jnp.zeros((1, K), jnp.int32))
                for _ in range(CU)]
        for k in range(K):
            sk = lane32 == k
            gs = [jnp.min(Rs[u], axis=(0, 1), keepdims=True) for u in range(CU)]
            vvecs = [jnp.min(jnp.where(Rs[u] == gs[u], lane16, NREG),
                             axis=(0, 1), keepdims=True) for u in range(CU)]
            vss = [vvecs[u][0, 0] for u in range(CU)]
            blks = [d_scrs[u][pl.ds(vss[u] * 8, 8), :] for u in range(CU)]
            rels = [jnp.min(jnp.where(blks[u] == gs[u], flatblk, BIG),
                            axis=(0, 1), keepdims=True) for u in range(CU)]
            fids = [rels[u] + vvecs[u] * (8 * NL) for u in range(CU)]
            blks = [jnp.where(flatblk == rels[u], inf, blks[u])
                    for u in range(CU)]
            for u in range(CU):
                d_scrs[u][pl.ds(vss[u] * 8, 8), :] = blks[u]
            nms = [jnp.min(blks[u], axis=(0, 1), keepdims=True)
                   for u in range(CU)]
            for u in range(CU):
                acc_val, acc_idx = accs[u]
                accs[u] = (jnp.where(sk, gs[u], acc_val),
                           jnp.where(sk, fids[u], acc_idx))
                Rs[u] = jnp.where(lane16 == vvecs[u], nms[u], Rs[u])
        for u in range(CU):
            c = cg * CU + u
            oval_ref[0, pl.ds(c, 1), :] = accs[u][0]
            oidx_ref[0, pl.ds(c, 1), :] = accs[u][1]
        return carry

    jax.lax.fori_loop(0, QB // CU, body, 0, unroll=False)


def _ball_query_pallas(pts, new_xyz):
    cx = new_xyz[:, :, 0:1]
    cy = new_xyz[:, :, 1:2]
    cz = new_xyz[:, :, 2:3]
    vals, idx = pl.pallas_call(
        _bq_kernel,
        grid=(B, NPOINT // QB),
        in_specs=[
            pl.BlockSpec((1, 3, NR, NL), lambda b, q: (b, 0, 0, 0)),
            pl.BlockSpec((1, QB, 1), lambda b, q: (b, q, 0)),
            pl.BlockSpec((1, QB, 1), lambda b, q: (b, q, 0)),
            pl.BlockSpec((1, QB, 1), lambda b, q: (b, q, 0)),
        ],
        out_specs=[
            pl.BlockSpec((1, QB, K), lambda b, q: (b, q, 0)),
            pl.BlockSpec((1, QB, K), lambda b, q: (b, q, 0)),
        ],
        out_shape=[
            jax.ShapeDtypeStruct((B, NPOINT, K), jnp.float32),
            jax.ShapeDtypeStruct((B, NPOINT, K), jnp.int32),
        ],
        scratch_shapes=[pltpu.VMEM((NR, NL), jnp.float32)] * CU,
        compiler_params=pltpu.CompilerParams(
            dimension_semantics=("parallel", "parallel"),
        ),
    )(pts, cx, cy, cz)
    idx32 = jnp.where(jnp.isinf(vals), -1, idx)
    n04 = jnp.sum((vals <= THR1).astype(jnp.int32), axis=-1, keepdims=True)
    s16 = jnp.arange(16, dtype=jnp.int32)[None, None, :]
    idx16 = jnp.where(s16 < n04, idx[:, :, :16], -1)
    return idx16, idx32


def _bn_relu(x, gamma, beta, eps=1e-5):
    mean = jnp.mean(x, axis=(0, 2, 3), keepdims=True)
    var = jnp.mean((x - mean) ** 2, axis=(0, 2, 3), keepdims=True)
    y = (x - mean) / jnp.sqrt(var + eps)
    y = y * gamma[None, :, None, None] + beta[None, :, None, None]
    return jax.nn.relu(y)


def _mlp_jax(g1, g2, params):
    outs = []
    for i, grouped in enumerate([g1, g2]):
        x = jnp.transpose(grouped, (0, 3, 1, 2))
        for j in range(len(MLPS[i]) - 1):
            W = params['W%d_%d' % (i, j)]
            x = jnp.einsum('oi,biqs->boqs', W, x)
            x = _bn_relu(x, params['gamma%d_%d' % (i, j)], params['beta%d_%d' % (i, j)])
        outs.append(jnp.max(x, axis=-1))
    return jnp.concatenate(outs, axis=1)


def _group(xyz, feat_NC, new_xyz, idx):
    idx_c = jnp.clip(idx, 0, None)
    grouped_xyz = jnp.take_along_axis(xyz[:, None, :, :], idx_c[:, :, :, None], axis=2)
    grouped_xyz = grouped_xyz - new_xyz[:, :, None, :]
    invalid = (idx < 0)[..., None]
    grouped_xyz = jnp.where(invalid, 0.0, grouped_xyz)
    grouped_feat = jnp.take_along_axis(feat_NC[:, None, :, :], idx_c[:, :, :, None], axis=2)
    grouped_feat = jnp.where(invalid, 0.0, grouped_feat)
    return jnp.concatenate([grouped_feat, grouped_xyz], axis=-1)


def _forward_core(xyz, features, params, new_xyz, idxs):
    feat_NC = jnp.transpose(features, (0, 2, 1))
    g1 = _group(xyz, feat_NC, new_xyz, idxs[0])
    g2 = _group(xyz, feat_NC, new_xyz, idxs[1])
    return _mlp_jax(g1, g2, params)


def kernel(xyz, features, W0_0, gamma0_0, beta0_0, W0_1, gamma0_1, beta0_1,
           W1_0, gamma1_0, beta1_0, W1_1, gamma1_1, beta1_1):
    params = {
        'W0_0': W0_0, 'gamma0_0': gamma0_0, 'beta0_0': beta0_0,
        'W0_1': W0_1, 'gamma0_1': gamma0_1, 'beta0_1': beta0_1,
        'W1_0': W1_0, 'gamma1_0': gamma1_0, 'beta1_0': beta1_0,
        'W1_1': W1_1, 'gamma1_1': gamma1_1, 'beta1_1': beta1_1,
    }
    pts = jnp.transpose(xyz, (0, 2, 1)).reshape(B, 3, NR, NL)
    new_xyz = _fps_pallas(pts)
    idx16, idx32 = _ball_query_pallas(pts, new_xyz)
    new_features = _forward_core(xyz, features, params, new_xyz, [idx16, idx32])
    return (new_xyz, new_features)
